# baseline (speedup 1.0000x reference)
import functools

import jax
import jax.numpy as jnp
from jax import lax
from jax.experimental import pallas as pl
from jax.experimental.pallas import tpu as pltpu
from jax.experimental.pallas import tpu_sc as plsc

QUEUE_ROWS = 32768
D = 128
ROWS_PER_TILE = 1024
CHUNK = 256
N_CHUNKS = ROWS_PER_TILE // CHUNK
NBUF = 2


def _copy_region(src_hbm, src_base, out_hbm, out_base, bufs, sems_in, sems_out):
    copies_in = []
    copies_out = []
    for c in range(N_CHUNKS):
        b = c % NBUF
        copies_in.append(pltpu.make_async_copy(
            src_hbm.at[pl.ds(src_base + c * CHUNK, CHUNK)], bufs[b], sems_in[b]))
        copies_out.append(pltpu.make_async_copy(
            bufs[b], out_hbm.at[pl.ds(out_base + c * CHUNK, CHUNK)], sems_out[b]))
    for c in range(NBUF):
        copies_in[c].start()
    for c in range(N_CHUNKS):
        copies_in[c].wait()
        copies_out[c].start()
        nxt = c + NBUF
        if nxt < N_CHUNKS:
            copies_out[c].wait()
            copies_in[nxt].start()
    for c in range(N_CHUNKS - NBUF, N_CHUNKS):
        copies_out[c].wait()


def _fifo_sc(x_hbm, q_hbm, out_hbm, buf0, buf1, si0, si1, so0, so1):
    nc = 2
    wid = lax.axis_index("s") * nc + lax.axis_index("c")
    out_base = wid * ROWS_PER_TILE
    shift = 4096
    n_q_tiles = (QUEUE_ROWS - shift) // ROWS_PER_TILE
    bufs = [buf0, buf1]
    sems_in = [si0, si1]
    sems_out = [so0, so1]

    @pl.when(wid < n_q_tiles)
    def _():
        _copy_region(q_hbm, out_base + shift, out_hbm, out_base,
                     bufs, sems_in, sems_out)

    @pl.when(wid >= n_q_tiles)
    def _():
        _copy_region(x_hbm, out_base - (QUEUE_ROWS - shift), out_hbm, out_base,
                     bufs, sems_in, sems_out)


def kernel(x, queue):
    mesh = plsc.VectorSubcoreMesh(core_axis_name="c", subcore_axis_name="s")
    k = functools.partial(
        pl.kernel,
        mesh=mesh,
        out_type=jax.ShapeDtypeStruct(queue.shape, queue.dtype),
        scratch_types=[
            pltpu.VMEM((CHUNK, D), jnp.float32),
            pltpu.VMEM((CHUNK, D), jnp.float32),
            pltpu.SemaphoreType.DMA,
            pltpu.SemaphoreType.DMA,
            pltpu.SemaphoreType.DMA,
            pltpu.SemaphoreType.DMA,
        ],
    )(_fifo_sc)
    return k(x, queue)


# speedup vs baseline: 2.2409x; 2.2409x over previous
import functools

import jax
import jax.numpy as jnp
from jax.experimental import pallas as pl
from jax.experimental.pallas import tpu as pltpu

QUEUE_ROWS = 32768
BLOCK = 4096


def _fifo_copy(q_ref, x_ref, o_ref):
    i = pl.program_id(0)

    @pl.when(i == 0)
    def _():
        o_ref[...] = x_ref[...]

    @pl.when(i > 0)
    def _():
        o_ref[...] = q_ref[...]


def kernel(x, queue):
    shift = x.shape[0]
    assert shift == BLOCK and QUEUE_ROWS % BLOCK == 0
    n_blocks = QUEUE_ROWS // BLOCK

    return pl.pallas_call(
        _fifo_copy,
        grid=(n_blocks,),
        in_specs=[
            pl.BlockSpec(
                (BLOCK, queue.shape[1]),
                lambda i: (jnp.maximum(i, 1), 0),
            ),
            pl.BlockSpec((BLOCK, x.shape[1]), lambda i: (0, 0)),
        ],
        out_specs=pl.BlockSpec(
            (BLOCK, queue.shape[1]), lambda i: ((i + n_blocks - 1) % n_blocks, 0)
        ),
        out_shape=jax.ShapeDtypeStruct(queue.shape, queue.dtype),
        compiler_params=pltpu.CompilerParams(
            dimension_semantics=("arbitrary",),
        ),
    )(queue, x)


# speedup vs baseline: 2.3574x; 1.0520x over previous
import jax
import jax.numpy as jnp
from jax.experimental import pallas as pl
from jax.experimental.pallas import tpu as pltpu

QUEUE_ROWS = 32768
BLOCK = 4096


def _fifo_copy(q_ref, x_ref, o_ref, *, n_q_blocks):
    i = pl.program_id(0)

    @pl.when(i < n_q_blocks)
    def _():
        o_ref[...] = q_ref[...]

    @pl.when(i >= n_q_blocks)
    def _():
        o_ref[...] = x_ref[...]


def kernel(x, queue):
    import functools

    shift = x.shape[0]
    assert shift % BLOCK == 0 and QUEUE_ROWS % BLOCK == 0
    n_blocks = QUEUE_ROWS // BLOCK
    n_x_blocks = shift // BLOCK
    n_q_blocks = n_blocks - n_x_blocks
    shift_blocks = shift // BLOCK

    return pl.pallas_call(
        functools.partial(_fifo_copy, n_q_blocks=n_q_blocks),
        grid=(n_blocks,),
        in_specs=[
            pl.BlockSpec(
                (BLOCK, queue.shape[1]),
                lambda i: (jnp.minimum(i + shift_blocks, n_blocks - 1), 0),
            ),
            pl.BlockSpec(
                (BLOCK, x.shape[1]),
                lambda i: (jnp.clip(i - n_q_blocks, 0, n_x_blocks - 1), 0),
            ),
        ],
        out_specs=pl.BlockSpec((BLOCK, queue.shape[1]), lambda i: (i, 0)),
        out_shape=jax.ShapeDtypeStruct(queue.shape, queue.dtype),
        compiler_params=pltpu.CompilerParams(
            dimension_semantics=("arbitrary",),
        ),
    )(queue, x)
